# Initial kernel scaffold; baseline (speedup 1.0000x reference)
import functools

import jax
import jax.numpy as jnp
from jax import lax
from jax.experimental import pallas as pl
from jax.experimental.pallas import tpu as pltpu
from jax.experimental.pallas import tpu_sc as plsc

N = 10000
E = 320000
D = 128
H = 128
R = 8
NB = 4
C = 32
G = 16
NR = N * R

NC = 2
NS = 16
L = 16
NW = NC * NS
EPW = E // NW
GRP = 16
NGRP = EPW // GRP
ROWS_PS = N // NS
DEG_PS = NR // NS

_mesh = plsc.VectorSubcoreMesh(core_axis_name="c", subcore_axis_name="s")


@functools.partial(
    pl.kernel,
    out_type=jax.ShapeDtypeStruct((NC * NR,), jnp.float32),
    mesh=_mesh,
    scratch_types=[
        pltpu.VMEM((NR,), jnp.float32),
        pltpu.VMEM((GRP,), jnp.int32),
        pltpu.VMEM((GRP,), jnp.int32),
        pltpu.VMEM_SHARED((NR,), jnp.float32),
    ],
)
def _deg_kernel(dst_hbm, rel_hbm, deg_out, table_v, dstbuf, relbuf, deg_sh):
    cid = lax.axis_index("c")
    sid = lax.axis_index("s")
    wid = sid * NC + cid

    zeros = jnp.zeros((L,), jnp.float32)

    @pl.loop(0, NR // L)
    def _zero(i):
        table_v[pl.ds(i * L, L)] = zeros

    @pl.when(sid == 0)
    def _init_shared():
        pltpu.sync_copy(table_v, deg_sh)

    plsc.subcore_barrier()

    base = pl.multiple_of(wid * EPW, GRP)
    ones = jnp.ones((L,), jnp.float32)

    @pl.loop(0, NGRP)
    def _accum(g):
        off = pl.multiple_of(base + g * GRP, GRP)
        pltpu.sync_copy(dst_hbm.at[pl.ds(off, GRP)], dstbuf)
        pltpu.sync_copy(rel_hbm.at[pl.ds(off, GRP)], relbuf)
        key = dstbuf[...] * R + relbuf[...]
        plsc.addupdate_scatter(table_v, [key], ones)

    pltpu.sync_copy(table_v, deg_sh, add=True)
    plsc.subcore_barrier()

    row0 = pl.multiple_of(cid * NR + sid * DEG_PS, 8)
    pltpu.sync_copy(deg_sh.at[pl.ds(sid * DEG_PS, DEG_PS)],
                    deg_out.at[pl.ds(row0, DEG_PS)])


@functools.partial(
    pl.kernel,
    out_type=jax.ShapeDtypeStruct((NC * N, H), jnp.float32),
    mesh=_mesh,
    scratch_types=[
        pltpu.VMEM((NR,), jnp.float32),
        pltpu.VMEM((R * NB,), jnp.float32),
        pltpu.VMEM((GRP, NB * D), jnp.float32),
        pltpu.VMEM((GRP, D), jnp.float32),
        pltpu.VMEM((GRP,), jnp.int32),
        pltpu.VMEM((GRP,), jnp.int32),
        pltpu.VMEM((GRP,), jnp.int32),
        pltpu.VMEM((NB, GRP), jnp.float32),
        pltpu.VMEM((125, D), jnp.float32),
        pltpu.VMEM_SHARED((N, H), jnp.float32),
        pltpu.SemaphoreType.DMA,
    ],
)
def _edge_kernel(src_hbm, dst_hbm, rel_hbm, norm_hbm, comp_hbm, z_hbm,
                 agg_out, norm_v, comp_v, zbuf, msgbuf, srcbuf, dstbuf,
                 relbuf, cbuf, zchunk, agg_sh, sem):
    cid = lax.axis_index("c")
    sid = lax.axis_index("s")
    wid = sid * NC + cid

    zeros = jnp.zeros((L,), jnp.float32)

    @pl.loop(0, 125 * D // L)
    def _zero(i):
        r = i // (D // L)
        c = i % (D // L)
        zchunk[r, pl.ds(c * L, L)] = zeros

    @pl.loop(0, ROWS_PS // 125)
    def _init(j):
        pltpu.sync_copy(zchunk, agg_sh.at[pl.ds(sid * ROWS_PS + j * 125, 125)])

    pltpu.sync_copy(norm_hbm, norm_v)
    pltpu.sync_copy(comp_hbm, comp_v)
    plsc.subcore_barrier()

    base = pl.multiple_of(wid * EPW, GRP)

    @pl.loop(0, NGRP)
    def _edges(g):
        off = pl.multiple_of(base + g * GRP, GRP)
        pltpu.sync_copy(src_hbm.at[pl.ds(off, GRP)], srcbuf)
        pltpu.sync_copy(dst_hbm.at[pl.ds(off, GRP)], dstbuf)
        pltpu.sync_copy(rel_hbm.at[pl.ds(off, GRP)], relbuf)

        pltpu.async_copy(z_hbm.at[srcbuf], zbuf, sem).wait()

        dst16 = dstbuf[...]
        rel16 = relbuf[...]
        key16 = dst16 * R + rel16
        w16 = plsc.load_gather(norm_v, [key16])
        for b in range(NB):
            cb = w16 * plsc.load_gather(comp_v, [rel16 * NB + b])
            cbuf[b, :] = cb

        for e in range(GRP):
            cs = [cbuf[b, e] for b in range(NB)]
            for j in range(D // L):
                acc = cs[0] * zbuf[e, pl.ds(0 * D + j * L, L)]
                acc += cs[1] * zbuf[e, pl.ds(1 * D + j * L, L)]
                acc += cs[2] * zbuf[e, pl.ds(2 * D + j * L, L)]
                acc += cs[3] * zbuf[e, pl.ds(3 * D + j * L, L)]
                msgbuf[e, pl.ds(j * L, L)] = acc

        pltpu.sync_copy(msgbuf, agg_sh.at[dstbuf], add=True)

    plsc.subcore_barrier()
    row0 = pl.multiple_of(cid * N + sid * ROWS_PS, 8)
    pltpu.sync_copy(agg_sh.at[pl.ds(sid * ROWS_PS, ROWS_PS)],
                    agg_out.at[pl.ds(row0, ROWS_PS)])


_ZBLK = 400
_ZGRID = N // _ZBLK
_DBLK = NR // _ZGRID


def _z_body(x_ref, bcat_ref, deg_ref, z_ref, norm_ref):
    z_ref[...] = jnp.dot(x_ref[...], bcat_ref[...],
                         preferred_element_type=jnp.float32)
    d = deg_ref[0, :] + deg_ref[1, :]
    norm_ref[...] = 1.0 / jnp.maximum(d, 1.0)


def _z_call(x, bcat, deg2):
    return pl.pallas_call(
        _z_body,
        grid=(_ZGRID,),
        in_specs=[
            pl.BlockSpec((_ZBLK, D), lambda i: (i, 0)),
            pl.BlockSpec((D, NB * H), lambda i: (0, 0)),
            pl.BlockSpec((NC, _DBLK), lambda i: (0, i)),
        ],
        out_specs=[
            pl.BlockSpec((_ZBLK, NB * H), lambda i: (i, 0)),
            pl.BlockSpec((_DBLK,), lambda i: (i,)),
        ],
        out_shape=[
            jax.ShapeDtypeStruct((N, NB * H), jnp.float32),
            jax.ShapeDtypeStruct((NR,), jnp.float32),
        ],
    )(x, bcat, deg2)


_FBLK = 400
_FGRID = N // _FBLK


def _f_body(a0_ref, a1_ref, x_ref, root_ref, bias_ref, batch_ref, fcw_ref,
            fcb_ref, out_ref, sum_acc, cnt_acc):
    i = pl.program_id(0)

    @pl.when(i == 0)
    def _init():
        sum_acc[...] = jnp.zeros_like(sum_acc)
        cnt_acc[...] = jnp.zeros_like(cnt_acc)

    h = a0_ref[...] + a1_ref[...] + jnp.dot(
        x_ref[...], root_ref[...], preferred_element_type=jnp.float32)
    h = jnp.maximum(h + bias_ref[0, :][None, :], 0.0)

    b = batch_ref[0, 0, :]
    gids = lax.broadcasted_iota(jnp.int32, (G, _FBLK), 0)
    onehot = (b[None, :] == gids).astype(jnp.float32)
    sum_acc[...] += jnp.dot(onehot, h, preferred_element_type=jnp.float32)
    cnt_acc[...] += jnp.broadcast_to(
        jnp.sum(onehot, axis=1, keepdims=True), (G, H))

    @pl.when(i == _FGRID - 1)
    def _final():
        pooled = sum_acc[...] / jnp.maximum(cnt_acc[...], 1.0)
        out_ref[...] = jnp.dot(pooled, fcw_ref[...],
                               preferred_element_type=jnp.float32) \
            + fcb_ref[0, :][None, :]


def _f_call(a0, a1, x, root, bias2, batch3, fc_w, fcb2):
    return pl.pallas_call(
        _f_body,
        grid=(_FGRID,),
        in_specs=[
            pl.BlockSpec((_FBLK, H), lambda i: (i, 0)),
            pl.BlockSpec((_FBLK, H), lambda i: (i, 0)),
            pl.BlockSpec((_FBLK, D), lambda i: (i, 0)),
            pl.BlockSpec((D, H), lambda i: (0, 0)),
            pl.BlockSpec((1, H), lambda i: (0, 0)),
            pl.BlockSpec((1, 1, _FBLK), lambda i: (i, 0, 0)),
            pl.BlockSpec((H, C), lambda i: (0, 0)),
            pl.BlockSpec((1, C), lambda i: (0, 0)),
        ],
        out_specs=pl.BlockSpec((G, C), lambda i: (0, 0)),
        out_shape=jax.ShapeDtypeStruct((G, C), jnp.float32),
        scratch_shapes=[
            pltpu.VMEM((G, H), jnp.float32),
            pltpu.VMEM((G, H), jnp.float32),
        ],
    )(a0, a1, x, root, bias2, batch3, fc_w, fcb2)


def kernel(x, edge_index, edge_type, edge_attr, batch, bases, comp, root,
           bias, fc_w, fc_b):
    del edge_attr
    src = edge_index[0].astype(jnp.int32)
    dst = edge_index[1].astype(jnp.int32)
    rel = edge_type.astype(jnp.int32)
    batch3 = batch.astype(jnp.int32).reshape(_FGRID, 1, _FBLK)
    bcat = jnp.transpose(bases, (1, 0, 2)).reshape(D, NB * H)
    comp_flat = comp.reshape(R * NB)

    deg = _deg_kernel(dst, rel)
    z, norm = _z_call(x, bcat, deg.reshape(NC, NR))
    agg = _edge_kernel(src, dst, rel, norm, comp_flat, z)
    a0 = agg[:N]
    a1 = agg[N:]
    out = _f_call(a0, a1, x, root, bias.reshape(1, H), batch3, fc_w,
                  fc_b.reshape(1, C))
    return out


# speedup vs baseline: 6.2501x; 6.2501x over previous
import functools

import jax
import jax.numpy as jnp
from jax import lax
from jax.experimental import pallas as pl
from jax.experimental.pallas import tpu as pltpu
from jax.experimental.pallas import tpu_sc as plsc

N = 10000
E = 320000
D = 128
H = 128
R = 8
NB = 4
C = 32
G = 16
NR = N * R

NC = 2
NS = 16
L = 16
NW = NC * NS
EPW = E // NW
GRP = 16
NGRP = EPW // GRP
ROWS_PS = N // NS
SRC_BITS = 14
SRC_MOD = 1 << SRC_BITS

_mesh = plsc.VectorSubcoreMesh(core_axis_name="c", subcore_axis_name="s")


DEG_COLS = 5000
DEG_PAD = 5120


@functools.partial(
    pl.kernel,
    out_type=jax.ShapeDtypeStruct((NW, L, DEG_PAD), jnp.float32),
    mesh=_mesh,
    scratch_types=[
        pltpu.VMEM((L, DEG_PAD), jnp.float32),
        pltpu.VMEM((GRP,), jnp.int32),
    ],
    compiler_params=pltpu.CompilerParams(use_tc_tiling_on_sc=False, needs_layout_passes=False),
)
def _deg_kernel(ep_hbm, deg_out, table_v, pbuf):
    cid = lax.axis_index("c")
    sid = lax.axis_index("s")
    wid = sid * NC + cid

    zeros = jnp.zeros((L,), jnp.float32)

    for r in range(L):
        @pl.loop(0, DEG_PAD // L)
        def _zero(i, r=r):
            table_v[r, pl.ds(i * L, L)] = zeros

    base = pl.multiple_of(wid * EPW, GRP)
    ones = jnp.ones((L,), jnp.float32)

    @pl.loop(0, NGRP)
    def _accum(g):
        off = pl.multiple_of(base + g * GRP, GRP)
        pltpu.sync_copy(ep_hbm.at[pl.ds(off, GRP)], pbuf)
        key16 = pbuf[...] >> SRC_BITS
        rel16 = key16 & (R - 1)
        dst16 = key16 >> 3
        row = (rel16 << 1) + (dst16 & 1)
        col = dst16 >> 1
        plsc.addupdate_scatter(table_v, [row, col], ones)

    pltpu.sync_copy(table_v, deg_out.at[wid])


HH = H // 2
EPS = E // NS
NGRP2 = EPS // GRP


@functools.partial(
    pl.kernel,
    out_type=jax.ShapeDtypeStruct((NC * N, HH), jnp.bfloat16),
    mesh=_mesh,
    scratch_types=[
        pltpu.VMEM((L, DEG_PAD), jnp.float32),
        pltpu.VMEM((R * NB,), jnp.float32),
        pltpu.VMEM((GRP, NB * HH), jnp.float32),
        pltpu.VMEM((GRP, HH), jnp.bfloat16),
        pltpu.VMEM((GRP,), jnp.int32),
        pltpu.VMEM((GRP,), jnp.int32),
        pltpu.VMEM((GRP,), jnp.int32),
        pltpu.VMEM((NB * GRP,), jnp.float32),
        pltpu.VMEM((125, HH), jnp.bfloat16),
        pltpu.VMEM_SHARED((N, HH), jnp.bfloat16),
        pltpu.SemaphoreType.DMA,
    ],
    compiler_params=pltpu.CompilerParams(use_tc_tiling_on_sc=False, needs_layout_passes=False),
)
def _edge_kernel(ep_hbm, norm_hbm, comp_hbm, z_hbm,
                 agg_out, norm_v, comp_v, zbuf, msgbuf, pbuf, srcbuf, dstbuf,
                 cbuf, zchunk, agg_sh, sem):
    cid = lax.axis_index("c")
    sid = lax.axis_index("s")

    zeros = jnp.zeros((2 * L,), jnp.bfloat16)

    @pl.loop(0, 125)
    def _zero(i):
        for c in range(HH // (2 * L)):
            zchunk[i, pl.ds(c * 2 * L, 2 * L)] = zeros

    @pl.loop(0, ROWS_PS // 125)
    def _init(j):
        pltpu.sync_copy(zchunk, agg_sh.at[pl.ds(sid * ROWS_PS + j * 125, 125)])

    pltpu.sync_copy(norm_hbm, norm_v)
    pltpu.sync_copy(comp_hbm, comp_v)
    plsc.subcore_barrier()

    base = pl.multiple_of(sid * EPS, GRP)

    @pl.loop(0, NGRP2)
    def _edges(g):
        off = pl.multiple_of(base + g * GRP, GRP)
        pltpu.sync_copy(ep_hbm.at[pl.ds(off, GRP)], pbuf)
        p16 = pbuf[...]
        key16 = p16 >> SRC_BITS
        dst16 = key16 >> 3
        srcbuf[...] = p16 & (SRC_MOD - 1)
        dstbuf[...] = dst16

        pltpu.async_copy(z_hbm.at[cid].at[srcbuf], zbuf, sem).wait()

        rel16 = key16 & (R - 1)
        w16 = plsc.load_gather(
            norm_v, [(rel16 << 1) + (dst16 & 1), dst16 >> 1])
        for b in range(NB):
            cbuf[pl.ds(b * GRP, GRP)] = (
                w16 * plsc.load_gather(comp_v, [rel16 * NB + b]))

        for e in range(GRP):
            cs = [plsc.load_gather(
                cbuf, [jnp.full((L,), b * GRP + e, jnp.int32)])
                for b in range(NB)]
            accs = []
            for j in range(HH // L):
                acc = cs[0] * zbuf[e, pl.ds(0 * HH + j * L, L)]
                acc += cs[1] * zbuf[e, pl.ds(1 * HH + j * L, L)]
                acc += cs[2] * zbuf[e, pl.ds(2 * HH + j * L, L)]
                acc += cs[3] * zbuf[e, pl.ds(3 * HH + j * L, L)]
                accs.append(acc)
            accs = [plsc.bitcast(plsc.bitcast(a, jnp.int32) + 32768,
                                 jnp.float32) for a in accs]
            for j in range(HH // (2 * L)):
                m = plsc.pack(accs[2 * j], accs[2 * j + 1],
                              format=plsc.PackFormat.INTERLEAVED)
                msgbuf[e, pl.ds(j * 2 * L, 2 * L)] = m

        pltpu.sync_copy(msgbuf, agg_sh.at[dstbuf], add=True)

    plsc.subcore_barrier()
    row0 = pl.multiple_of(cid * N + sid * ROWS_PS, 8)
    pltpu.sync_copy(agg_sh.at[pl.ds(sid * ROWS_PS, ROWS_PS)],
                    agg_out.at[pl.ds(row0, ROWS_PS)])


_ZBLK = 400
_ZGRID = N // _ZBLK
_DBLK = NR // _ZGRID


def _z_body(x_ref, bcat_ref, z_ref):
    zc = jnp.dot(x_ref[...], bcat_ref[...],
                 preferred_element_type=jnp.float32)
    z_ref[0] = zc[:, :NB * HH]
    z_ref[1] = zc[:, NB * HH:]


def _z_call(x, bcat):
    return pl.pallas_call(
        _z_body,
        grid=(_ZGRID,),
        in_specs=[
            pl.BlockSpec((_ZBLK, D), lambda i: (i, 0)),
            pl.BlockSpec((D, NB * H), lambda i: (0, 0)),
        ],
        out_specs=pl.BlockSpec((NC, _ZBLK, NB * HH), lambda i: (0, i, 0)),
        out_shape=jax.ShapeDtypeStruct((NC, N, NB * HH), jnp.float32),
    )(x, bcat)


_NBLK = 128
_NGRID = DEG_PAD // _NBLK


def _n_body(deg_ref, norm_ref):
    s = jnp.sum(deg_ref[...], axis=0)
    norm_ref[...] = 1.0 / jnp.maximum(s, 1.0)


def _n_call(deg_parts):
    return pl.pallas_call(
        _n_body,
        grid=(_NGRID,),
        in_specs=[pl.BlockSpec((NW, L, _NBLK), lambda i: (0, 0, i))],
        out_specs=pl.BlockSpec((L, _NBLK), lambda i: (0, i)),
        out_shape=jax.ShapeDtypeStruct((L, DEG_PAD), jnp.float32),
    )(deg_parts)


_FBLK = 400
_FGRID = N // _FBLK


def _f_body(a0_ref, a1_ref, x_ref, root_ref, bias_ref, batch_ref, fcw_ref,
            fcb_ref, out_ref, sum_acc, cnt_acc):
    i = pl.program_id(0)

    @pl.when(i == 0)
    def _init():
        sum_acc[...] = jnp.zeros_like(sum_acc)
        cnt_acc[...] = jnp.zeros_like(cnt_acc)

    h = jnp.concatenate([a0_ref[...], a1_ref[...]], axis=1) + jnp.dot(
        x_ref[...], root_ref[...], preferred_element_type=jnp.float32)
    h = jnp.maximum(h + bias_ref[0, :][None, :], 0.0)

    b = batch_ref[0, 0, :]
    gids = lax.broadcasted_iota(jnp.int32, (G, _FBLK), 0)
    onehot = (b[None, :] == gids).astype(jnp.float32)
    sum_acc[...] += jnp.dot(onehot, h, preferred_element_type=jnp.float32)
    cnt_acc[...] += jnp.broadcast_to(
        jnp.sum(onehot, axis=1, keepdims=True), (G, H))

    @pl.when(i == _FGRID - 1)
    def _final():
        pooled = sum_acc[...] / jnp.maximum(cnt_acc[...], 1.0)
        out_ref[...] = jnp.dot(pooled, fcw_ref[...],
                               preferred_element_type=jnp.float32) \
            + fcb_ref[0, :][None, :]


def _f_call(a0, a1, x, root, bias2, batch3, fc_w, fcb2):
    return pl.pallas_call(
        _f_body,
        grid=(_FGRID,),
        in_specs=[
            pl.BlockSpec((_FBLK, HH), lambda i: (i, 0)),
            pl.BlockSpec((_FBLK, HH), lambda i: (i, 0)),
            pl.BlockSpec((_FBLK, D), lambda i: (i, 0)),
            pl.BlockSpec((D, H), lambda i: (0, 0)),
            pl.BlockSpec((1, H), lambda i: (0, 0)),
            pl.BlockSpec((1, 1, _FBLK), lambda i: (i, 0, 0)),
            pl.BlockSpec((H, C), lambda i: (0, 0)),
            pl.BlockSpec((1, C), lambda i: (0, 0)),
        ],
        out_specs=pl.BlockSpec((G, C), lambda i: (0, 0)),
        out_shape=jax.ShapeDtypeStruct((G, C), jnp.float32),
        scratch_shapes=[
            pltpu.VMEM((G, H), jnp.float32),
            pltpu.VMEM((G, H), jnp.float32),
        ],
    )(a0, a1, x, root, bias2, batch3, fc_w, fcb2)


def kernel(x, edge_index, edge_type, edge_attr, batch, bases, comp, root,
           bias, fc_w, fc_b):
    del edge_attr
    src = edge_index[0].astype(jnp.int32)
    dst = edge_index[1].astype(jnp.int32)
    rel = edge_type.astype(jnp.int32)
    epack = ((dst * R + rel) << SRC_BITS) | src
    batch3 = batch.astype(jnp.int32).reshape(_FGRID, 1, _FBLK)
    bcat = jnp.transpose(bases.reshape(NB, D, NC, HH),
                         (1, 2, 0, 3)).reshape(D, NB * H)
    comp_flat = comp.reshape(R * NB)

    deg_parts = _deg_kernel(epack)
    norm_tab = _n_call(deg_parts)
    z3 = _z_call(x, bcat)
    agg_bf = _edge_kernel(epack, norm_tab, comp_flat, z3)
    agg = jnp.transpose(agg_bf.reshape(NC * N, 2, L, 2),
                        (0, 1, 3, 2)).reshape(NC * N, HH)
    agg = agg.astype(jnp.float32)
    a0 = agg[:N]
    a1 = agg[N:]
    out = _f_call(a0, a1, x, root, bias.reshape(1, H), batch3, fc_w,
                  fc_b.reshape(1, C))
    return out


# speedup vs baseline: 16.2806x; 2.6049x over previous
import functools

import jax
import jax.numpy as jnp
from jax import lax
from jax.experimental import pallas as pl
from jax.experimental.pallas import tpu as pltpu
from jax.experimental.pallas import tpu_sc as plsc

N = 10000
E = 320000
D = 128
H = 128
R = 8
NB = 4
C = 32
G = 16
NR = N * R

NC = 2
NS = 16
L = 16
NW = NC * NS
EPW = E // NW
GRP = 16
NGRP = EPW // GRP
ROWS_PS = N // NS
SRC_BITS = 14
SRC_MOD = 1 << SRC_BITS

_mesh = plsc.VectorSubcoreMesh(core_axis_name="c", subcore_axis_name="s")


DEG_COLS = 5000
DEG_PAD = 5120


@functools.partial(
    pl.kernel,
    out_type=jax.ShapeDtypeStruct((NW, L, DEG_PAD), jnp.float32),
    mesh=_mesh,
    scratch_types=[
        pltpu.VMEM((L, DEG_PAD), jnp.float32),
        pltpu.VMEM((EPW,), jnp.int32),
    ],
    compiler_params=pltpu.CompilerParams(use_tc_tiling_on_sc=False, needs_layout_passes=False),
)
def _deg_kernel(ep_hbm, deg_out, table_v, ep_v):
    cid = lax.axis_index("c")
    sid = lax.axis_index("s")
    wid = sid * NC + cid

    zeros = jnp.zeros((L,), jnp.float32)

    for r in range(L):
        @pl.loop(0, DEG_PAD // L)
        def _zero(i, r=r):
            table_v[r, pl.ds(i * L, L)] = zeros

    base = pl.multiple_of(wid * EPW, GRP)
    ones = jnp.ones((L,), jnp.float32)
    pltpu.sync_copy(ep_hbm.at[pl.ds(base, EPW)], ep_v)

    @pl.loop(0, NGRP)
    def _accum(g):
        key16 = ep_v[pl.ds(g * GRP, GRP)] >> SRC_BITS
        rel16 = key16 & (R - 1)
        dst16 = key16 >> 3
        row = (rel16 << 1) + (dst16 & 1)
        col = dst16 >> 1
        plsc.addupdate_scatter(table_v, [row, col], ones)

    pltpu.sync_copy(table_v, deg_out.at[wid])


HH = H // 2
EPS = E // NS
NGRP2 = EPS // GRP
NCHUNK = 5
CGRP = NGRP2 // NCHUNK
CHUNK_W = EPS // NCHUNK


@functools.partial(
    pl.kernel,
    out_type=jax.ShapeDtypeStruct((NC * N, HH), jnp.bfloat16),
    mesh=_mesh,
    scratch_types=[
        pltpu.VMEM((L, DEG_PAD), jnp.float32),
        pltpu.VMEM((R * NB,), jnp.float32),
        pltpu.VMEM((EPS // NCHUNK,), jnp.int32),
        pltpu.VMEM((GRP, NB * HH), jnp.float32),
        pltpu.VMEM((GRP, NB * HH), jnp.float32),
        pltpu.VMEM((GRP, HH), jnp.bfloat16),
        pltpu.VMEM((GRP,), jnp.int32),
        pltpu.VMEM((GRP,), jnp.int32),
        pltpu.VMEM((GRP,), jnp.int32),
        pltpu.VMEM((NB * GRP,), jnp.float32),
        pltpu.VMEM((125, HH), jnp.bfloat16),
        pltpu.VMEM_SHARED((N, HH), jnp.bfloat16),
        pltpu.SemaphoreType.DMA,
        pltpu.SemaphoreType.DMA,
    ],
    compiler_params=pltpu.CompilerParams(use_tc_tiling_on_sc=False, needs_layout_passes=False),
)
def _edge_kernel(ep_hbm, norm_hbm, comp_hbm, z_hbm,
                 agg_out, norm_v, comp_v, ep_v, zbufa, zbufb, msgbuf,
                 srcbufa, srcbufb, dstbuf, cbuf, zchunk, agg_sh, sem0, sem1):
    cid = lax.axis_index("c")
    sid = lax.axis_index("s")

    zeros = jnp.zeros((2 * L,), jnp.bfloat16)

    @pl.loop(0, 125)
    def _zero(i):
        for c in range(HH // (2 * L)):
            zchunk[i, pl.ds(c * 2 * L, 2 * L)] = zeros

    @pl.loop(0, ROWS_PS // 125)
    def _init(j):
        pltpu.sync_copy(zchunk, agg_sh.at[pl.ds(sid * ROWS_PS + j * 125, 125)])

    pltpu.sync_copy(norm_hbm, norm_v)
    pltpu.sync_copy(comp_hbm, comp_v)
    plsc.subcore_barrier()

    bufs = ((zbufa, srcbufa, sem0), (zbufb, srcbufb, sem1))

    def _prefetch(g, par):
        zb, sb, sem = bufs[par]
        p16 = ep_v[pl.ds(g * GRP, GRP)]
        sb[...] = p16 & (SRC_MOD - 1)
        pltpu.async_copy(z_hbm.at[cid].at[sb], zb, sem)

    def _process(g, par):
        zb, sb, sem = bufs[par]
        p16 = ep_v[pl.ds(g * GRP, GRP)]
        key16 = p16 >> SRC_BITS
        dst16 = key16 >> 3
        rel16 = key16 & (R - 1)
        dstbuf[...] = dst16
        w16 = plsc.load_gather(
            norm_v, [(rel16 << 1) + (dst16 & 1), dst16 >> 1])
        for b in range(NB):
            cbuf[pl.ds(b * GRP, GRP)] = (
                w16 * plsc.load_gather(comp_v, [rel16 * NB + b]))

        pltpu.make_async_copy(z_hbm.at[cid].at[sb], zb, sem).wait()

        for e in range(GRP):
            cs = [plsc.load_gather(
                cbuf, [jnp.full((L,), b * GRP + e, jnp.int32)])
                for b in range(NB)]
            accs = []
            for j in range(HH // L):
                acc = cs[0] * zb[e, pl.ds(0 * HH + j * L, L)]
                acc += cs[1] * zb[e, pl.ds(1 * HH + j * L, L)]
                acc += cs[2] * zb[e, pl.ds(2 * HH + j * L, L)]
                acc += cs[3] * zb[e, pl.ds(3 * HH + j * L, L)]
                accs.append(acc)
            accs = [plsc.bitcast(plsc.bitcast(a, jnp.int32) + 32768,
                                 jnp.float32) for a in accs]
            for j in range(HH // (2 * L)):
                m = plsc.pack(accs[2 * j], accs[2 * j + 1],
                              format=plsc.PackFormat.INTERLEAVED)
                msgbuf[e, pl.ds(j * 2 * L, 2 * L)] = m

        pltpu.sync_copy(msgbuf, agg_sh.at[dstbuf], add=True)

    base = pl.multiple_of(sid * EPS, GRP)

    @pl.loop(0, NCHUNK)
    def _chunks(c):
        pltpu.sync_copy(ep_hbm.at[pl.ds(base + c * CHUNK_W, CHUNK_W)], ep_v)
        _prefetch(0, 0)

        @pl.loop(0, CGRP // 2)
        def _edges(g2):
            g = g2 * 2
            _prefetch(g + 1, 1)
            _process(g, 0)

            @pl.when(g2 < CGRP // 2 - 1)
            def _pf():
                _prefetch(g + 2, 0)

            _process(g + 1, 1)

    plsc.subcore_barrier()
    row0 = pl.multiple_of(cid * N + sid * ROWS_PS, 8)
    pltpu.sync_copy(agg_sh.at[pl.ds(sid * ROWS_PS, ROWS_PS)],
                    agg_out.at[pl.ds(row0, ROWS_PS)])


_ZBLK = 400
_ZGRID = N // _ZBLK
_DBLK = NR // _ZGRID


def _z_body(x_ref, bcat_ref, z_ref):
    zc = jnp.dot(x_ref[...], bcat_ref[...],
                 preferred_element_type=jnp.float32)
    z_ref[0] = zc[:, :NB * HH]
    z_ref[1] = zc[:, NB * HH:]


def _z_call(x, bcat):
    return pl.pallas_call(
        _z_body,
        grid=(_ZGRID,),
        in_specs=[
            pl.BlockSpec((_ZBLK, D), lambda i: (i, 0)),
            pl.BlockSpec((D, NB * H), lambda i: (0, 0)),
        ],
        out_specs=pl.BlockSpec((NC, _ZBLK, NB * HH), lambda i: (0, i, 0)),
        out_shape=jax.ShapeDtypeStruct((NC, N, NB * HH), jnp.float32),
    )(x, bcat)


_NBLK = 128
_NGRID = DEG_PAD // _NBLK


def _n_body(deg_ref, norm_ref):
    s = jnp.sum(deg_ref[...], axis=0)
    norm_ref[...] = 1.0 / jnp.maximum(s, 1.0)


def _n_call(deg_parts):
    return pl.pallas_call(
        _n_body,
        grid=(_NGRID,),
        in_specs=[pl.BlockSpec((NW, L, _NBLK), lambda i: (0, 0, i))],
        out_specs=pl.BlockSpec((L, _NBLK), lambda i: (0, i)),
        out_shape=jax.ShapeDtypeStruct((L, DEG_PAD), jnp.float32),
    )(deg_parts)


_FBLK = 400
_FGRID = N // _FBLK


def _f_body(a0_ref, a1_ref, x_ref, root_ref, bias_ref, batch_ref, fcw_ref,
            fcb_ref, out_ref, sum_acc, cnt_acc):
    i = pl.program_id(0)

    @pl.when(i == 0)
    def _init():
        sum_acc[...] = jnp.zeros_like(sum_acc)
        cnt_acc[...] = jnp.zeros_like(cnt_acc)

    h = jnp.concatenate([a0_ref[...], a1_ref[...]], axis=1) + jnp.dot(
        x_ref[...], root_ref[...], preferred_element_type=jnp.float32)
    h = jnp.maximum(h + bias_ref[0, :][None, :], 0.0)

    b = batch_ref[0, 0, :]
    gids = lax.broadcasted_iota(jnp.int32, (G, _FBLK), 0)
    onehot = (b[None, :] == gids).astype(jnp.float32)
    sum_acc[...] += jnp.dot(onehot, h, preferred_element_type=jnp.float32)
    cnt_acc[...] += jnp.broadcast_to(
        jnp.sum(onehot, axis=1, keepdims=True), (G, H))

    @pl.when(i == _FGRID - 1)
    def _final():
        pooled = sum_acc[...] / jnp.maximum(cnt_acc[...], 1.0)
        out_ref[...] = jnp.dot(pooled, fcw_ref[...],
                               preferred_element_type=jnp.float32) \
            + fcb_ref[0, :][None, :]


def _f_call(a0, a1, x, root, bias2, batch3, fc_w, fcb2):
    return pl.pallas_call(
        _f_body,
        grid=(_FGRID,),
        in_specs=[
            pl.BlockSpec((_FBLK, HH), lambda i: (i, 0)),
            pl.BlockSpec((_FBLK, HH), lambda i: (i, 0)),
            pl.BlockSpec((_FBLK, D), lambda i: (i, 0)),
            pl.BlockSpec((D, H), lambda i: (0, 0)),
            pl.BlockSpec((1, H), lambda i: (0, 0)),
            pl.BlockSpec((1, 1, _FBLK), lambda i: (i, 0, 0)),
            pl.BlockSpec((H, C), lambda i: (0, 0)),
            pl.BlockSpec((1, C), lambda i: (0, 0)),
        ],
        out_specs=pl.BlockSpec((G, C), lambda i: (0, 0)),
        out_shape=jax.ShapeDtypeStruct((G, C), jnp.float32),
        scratch_shapes=[
            pltpu.VMEM((G, H), jnp.float32),
            pltpu.VMEM((G, H), jnp.float32),
        ],
    )(a0, a1, x, root, bias2, batch3, fc_w, fcb2)


def kernel(x, edge_index, edge_type, edge_attr, batch, bases, comp, root,
           bias, fc_w, fc_b):
    del edge_attr
    src = edge_index[0].astype(jnp.int32)
    dst = edge_index[1].astype(jnp.int32)
    rel = edge_type.astype(jnp.int32)
    epack = ((dst * R + rel) << SRC_BITS) | src
    batch3 = batch.astype(jnp.int32).reshape(_FGRID, 1, _FBLK)
    bcat = jnp.transpose(bases.reshape(NB, D, NC, HH),
                         (1, 2, 0, 3)).reshape(D, NB * H)
    comp_flat = comp.reshape(R * NB)

    deg_parts = _deg_kernel(epack)
    norm_tab = _n_call(deg_parts)
    z3 = _z_call(x, bcat)
    agg_bf = _edge_kernel(epack, norm_tab, comp_flat, z3)
    agg = jnp.transpose(agg_bf.reshape(NC * N, 2, L, 2),
                        (0, 1, 3, 2)).reshape(NC * N, HH)
    agg = agg.astype(jnp.float32)
    a0 = agg[:N]
    a1 = agg[N:]
    out = _f_call(a0, a1, x, root, bias.reshape(1, H), batch3, fc_w,
                  fc_b.reshape(1, C))
    return out
